# baseline (speedup 1.0000x reference)
import functools
import numpy as np
import jax
import jax.numpy as jnp
from jax import lax
from jax.experimental import pallas as pl
from jax.experimental.pallas import tpu as pltpu
from jax.experimental.pallas import tpu_sc as plsc

N = 10000
E = 320000
D = 128
R = 8
V = 10000
T = 16
L = 16
NC = 2
NS = 16
NW = NC * NS
C = D // L
CPS = C // NC
NT = N * T
RN = R * N
CSH = RN // NS
AROWS = CPS * N
GROWS = R * N
TROWS = 1280
NTP = TROWS * 128
EROWS = 2560
EP = EROWS * 128
SROWS_E = EROWS // NS
SROWS_T = TROWS // NS
BLK = 40
NBUF = 8
PD = NBUF - 2

_SC_PARAMS = pltpu.CompilerParams(
    use_tc_tiling_on_sc=False, needs_layout_passes=False)


def _mesh():
    return plsc.VectorSubcoreMesh(
        core_axis_name="c", subcore_axis_name="s", num_cores=NC, num_subcores=NS
    )


def _row_range(total, sid):
    return (total * sid) // NS, (total * (sid + 1)) // NS


def _ring(table, gblk, sblk, rows_v, acc_sh, gsems, ssems):
    dg = {}
    pend = {}
    for j in range(min(PD, BLK)):
        s = j % NBUF
        dg[s] = pltpu.async_copy(table.at[gblk.at[j]], rows_v.at[s], gsems[s])
    for j in range(BLK):
        s = j % NBUF
        dg.pop(s).wait()
        pend[s] = pltpu.async_copy(rows_v.at[s], acc_sh.at[sblk.at[j]],
                                   ssems[s], add=True)
        nj = j + PD
        if nj < BLK:
            s2 = nj % NBUF
            if s2 in pend:
                pend.pop(s2).wait()
            dg[s2] = pltpu.async_copy(table.at[gblk.at[nj]], rows_v.at[s2],
                                      gsems[s2])
    for s2 in list(pend):
        pend.pop(s2).wait()


TBLK = TROWS // NC // NS


def _embed_body(emb_z, xpad_f, psidx, esidx_f, zeros3, zerosf,
                sp_out, cnt_out,
                gblk, sblk, rows_v, cnt_local, acc_sh, gsem, ssem):
    cid = lax.axis_index("c")
    sid = lax.axis_index("s")
    gsems = [gsem.at[i] for i in range(2)]
    ssems = [ssem.at[i] for i in range(2)]

    pltpu.sync_copy(zeros3, rows_v.at[0, pl.ds(0, 125)])
    for i in range(5):
        pltpu.sync_copy(rows_v.at[0, pl.ds(0, 125)],
                        acc_sh.at[pl.ds(625 * sid + 125 * i, 125)])

    @pl.when(sid == 0)
    def _():
        pltpu.sync_copy(rows_v.at[0, pl.ds(0, 16)], acc_sh.at[pl.ds(N, 16)])

    pltpu.sync_copy(zerosf, cnt_local)
    plsc.subcore_barrier()

    ones = jnp.full((L,), 1.0, jnp.float32)
    clo = sid * CSH
    half = EP // NC

    def _cnt_blk(b, carry):
        pltpu.sync_copy(esidx_f.at[pl.ds(cid * half + b * 5120, 5120)], gblk)

        def _cnt(k, c2):
            f = gblk[pl.ds(16 * k, 16)]
            fl = f - clo
            m = (fl >= 0) & (fl < CSH)
            fl = jnp.where(m, fl, 0)
            plsc.addupdate_scatter(cnt_local, [fl], ones, mask=m)
            return c2

        lax.fori_loop(0, 320, _cnt, 0)
        return carry

    lax.fori_loop(0, half // 5120, _cnt_blk, 0)

    row0 = cid * (TROWS // NC) + sid * TBLK
    pltpu.sync_copy(xpad_f.at[pl.ds(row0 * 128, TBLK * 128)], gblk)
    pltpu.sync_copy(psidx.at[pl.ds(row0, TBLK)], sblk)

    dg = {}
    pend = {}
    for j in range(2):
        dg[j] = pltpu.async_copy(
            emb_z.at[gblk.at[pl.ds(128 * j, 128)]], rows_v.at[j], gsems[j])
    for j in range(TBLK):
        s = j % 2
        dg.pop(s).wait()
        pend[s] = pltpu.async_copy(rows_v.at[s], acc_sh.at[sblk.at[j]],
                                   ssems[s], add=True)
        if j + 2 < TBLK:
            pend.pop(s).wait()
            dg[s] = pltpu.async_copy(
                emb_z.at[gblk.at[pl.ds(128 * (j + 2), 128)]], rows_v.at[s],
                gsems[s])
    for s in list(pend):
        pend.pop(s).wait()

    plsc.subcore_barrier()

    for i in range(5):
        base = 625 * sid + 125 * i
        pltpu.sync_copy(acc_sh.at[pl.ds(base, 125)],
                        rows_v.at[0, pl.ds(0, 125)])
        pltpu.sync_copy(rows_v.at[0, pl.ds(0, 125)],
                        sp_out.at[cid, pl.ds(base, 125)])
    pltpu.sync_copy(cnt_local, cnt_out.at[cid, sid])


def _sc_embed():
    return pl.kernel(
        _embed_body,
        out_type=(
            jax.ShapeDtypeStruct((NC, N, D), jnp.float32),
            jax.ShapeDtypeStruct((NC, NS, CSH), jnp.float32),
        ),
        mesh=_mesh(),
        scratch_types=[
            pltpu.VMEM((TBLK * 128,), jnp.int32),
            pltpu.VMEM((TBLK, 128), jnp.int32),
            pltpu.VMEM((2, 128, D), jnp.float32),
            pltpu.VMEM((CSH,), jnp.float32),
            pltpu.MemorySpace.VMEM_SHARED((N + 16, D), jnp.float32),
            pltpu.SemaphoreType.DMA((2,)),
            pltpu.SemaphoreType.DMA((2,)),
        ],
        compiler_params=_SC_PARAMS,
    )


def _agg_body(h_flat, gsrc, esidx, zeros2, agg_out,
              buf, gblk, sblk, rows_v, acc_sh, gsem, ssem):
    cid = lax.axis_index("c")
    sid = lax.axis_index("s")
    gsems = [gsem.at[i] for i in range(NBUF)]
    ssems = [ssem.at[i] for i in range(NBUF)]

    for lc in range(CPS):
        c = cid * CPS + lc
        pltpu.sync_copy(zeros2, buf)
        for i in range(8):
            pltpu.sync_copy(buf, acc_sh.at[pl.ds(5000 * sid + 625 * i, 625)])
        plsc.subcore_barrier()

        for blk in range(SROWS_E // BLK):
            row0 = sid * SROWS_E + blk * BLK
            pltpu.sync_copy(gsrc.at[c, pl.ds(row0, BLK)], gblk)
            pltpu.sync_copy(esidx.at[pl.ds(row0, BLK)], sblk)
            _ring(h_flat, gblk, sblk, rows_v, acc_sh, gsems, ssems)
        plsc.subcore_barrier()

        def _wb(i, carry):
            base = 5000 * sid + 625 * i
            pltpu.sync_copy(acc_sh.at[pl.ds(base, 625)], buf)
            pltpu.sync_copy(buf, agg_out.at[pl.ds(base, 625), c, :])
            return carry

        lax.fori_loop(0, 8, _wb, 0)
        plsc.subcore_barrier()


def _sc_agg():
    return pl.kernel(
        _agg_body,
        out_type=jax.ShapeDtypeStruct((GROWS, C, L), jnp.float32),
        mesh=_mesh(),
        scratch_types=[
            pltpu.VMEM((625, L), jnp.float32),
            pltpu.VMEM((BLK, 128), jnp.int32),
            pltpu.VMEM((BLK, 128), jnp.int32),
            pltpu.VMEM((NBUF, 128, L), jnp.float32),
            pltpu.MemorySpace.VMEM_SHARED((GROWS + 128, L), jnp.float32),
            pltpu.SemaphoreType.DMA((NBUF,)),
            pltpu.SemaphoreType.DMA((NBUF,)),
        ],
        compiler_params=_SC_PARAMS,
    )


BN2 = 2000


def _scale_body(x_ref, s_ref, out_ref):
    mask = (x_ref[...] != 0).astype(jnp.float32)
    den = jnp.sum(mask, axis=1, keepdims=True)
    rec = 1.0 / jnp.maximum(den, 1.0)
    out_ref[...] = (s_ref[0] + s_ref[1]) * rec


def _tc_scale():
    return pl.pallas_call(
        _scale_body,
        grid=(N // BN2,),
        in_specs=[
            pl.BlockSpec((BN2, T), lambda i: (i, 0)),
            pl.BlockSpec((NC, BN2, D), lambda i: (0, i, 0)),
        ],
        out_specs=pl.BlockSpec((BN2, D), lambda i: (i, 0)),
        out_shape=jax.ShapeDtypeStruct((N, D), jnp.float32),
    )


BN = 400


def _combine_body(h_ref, agg_ref, cnt_ref, wrel_ref, wroot_ref, b_ref,
                  res_ref, out_ref):
    h = h_ref[...]
    acc = jnp.dot(h, wroot_ref[...], preferred_element_type=jnp.float32)
    acc = acc + b_ref[...]
    cnt = jnp.sum(cnt_ref[...].reshape(BN, NC, R), axis=1)
    recip = 1.0 / jnp.maximum(cnt, 1.0)
    for r in range(R):
        ar = agg_ref[r] * recip[:, r][:, None]
        acc = acc + jnp.dot(ar, wrel_ref[r], preferred_element_type=jnp.float32)
    acc = jnp.maximum(acc, 0.0)
    if res_ref is not None:
        acc = acc + res_ref[...]
    out_ref[...] = acc


def _tc_combine(with_res):
    body = _combine_body if with_res else (
        lambda h, a, c, wr, wo, b, o: _combine_body(h, a, c, wr, wo, b, None, o)
    )
    in_specs = [
        pl.BlockSpec((BN, D), lambda i: (i, 0)),
        pl.BlockSpec((R, BN, D), lambda i: (0, i, 0)),
        pl.BlockSpec((BN, NC * R), lambda i: (i, 0)),
        pl.BlockSpec((R, D, D), lambda i: (0, 0, 0)),
        pl.BlockSpec((D, D), lambda i: (0, 0)),
        pl.BlockSpec((1, D), lambda i: (0, 0)),
    ]
    if with_res:
        in_specs.append(pl.BlockSpec((BN, D), lambda i: (i, 0)))
    return pl.pallas_call(
        body,
        grid=(N // BN,),
        in_specs=in_specs,
        out_specs=pl.BlockSpec((BN, D), lambda i: (i, 0)),
        out_shape=jax.ShapeDtypeStruct((N, D), jnp.float32),
    )


def _perm(h):
    return h.reshape(N, C, L).transpose(1, 0, 2).reshape(C * N, L)


def _unperm(hp):
    return hp.reshape(C, N, L).transpose(1, 0, 2).reshape(N, D)


def _unperm_agg(agg_out):
    return agg_out.reshape(C, R, N, L).transpose(1, 2, 0, 3).reshape(R, N, D)


def kernel(x, edge_index, edge_type, emb, W_rel0, W_root0, b0,
           W_rel1, W_root1, b1):
    x = x.astype(jnp.int32)
    src = edge_index[0].astype(jnp.int32)
    dst = edge_index[1].astype(jnp.int32)
    et = edge_type.astype(jnp.int32)

    emb_z = emb.at[0].set(0.0)
    xpad = jnp.concatenate(
        [x.reshape(NT), jnp.zeros((NTP - NT,), jnp.int32)]
    ).reshape(TROWS, 128)
    psidx = jnp.concatenate(
        [jnp.arange(NT, dtype=jnp.int32) // T,
         jnp.full((NTP - NT,), N, jnp.int32)]).reshape(TROWS, 128)
    esidx_f = jnp.concatenate(
        [et * N + dst, jnp.full((EP - E,), GROWS, jnp.int32)])
    esidx = esidx_f.reshape(EROWS, 128)
    src_p = jnp.concatenate([src, jnp.zeros((EP - E,), jnp.int32)])
    gsrc = (src_p[None, :] * C + jnp.arange(C, dtype=jnp.int32)[:, None]
            ).reshape(C, EROWS, 128)
    zeros2 = jnp.zeros((625, L), jnp.float32)
    zeros3 = jnp.zeros((125, D), jnp.float32)
    zerosf = jnp.zeros((CSH,), jnp.float32)
    b0r = b0.reshape(1, D)
    b1r = b1.reshape(1, D)

    sp_p, cnt_raw = _sc_embed()(emb_z, xpad.reshape(NTP), psidx,
                                esidx_f, zeros3, zerosf)
    cnt_t = cnt_raw.reshape(NC, R, N).transpose(2, 0, 1).reshape(N, NC * R)

    h0 = _tc_scale()(x, sp_p)

    agg0 = _sc_agg()(h0.reshape(N * C, L), gsrc, esidx, zeros2
                     ).reshape(R, N, D)
    out0 = _tc_combine(False)(h0, agg0, cnt_t, W_rel0, W_root0, b0r)

    agg1 = _sc_agg()(out0.reshape(N * C, L), gsrc, esidx, zeros2
                     ).reshape(R, N, D)
    out = _tc_combine(True)(out0, agg1, cnt_t, W_rel1, W_root1, b1r, out0)
    return out


# speedup vs baseline: 1.0136x; 1.0136x over previous
import functools
import numpy as np
import jax
import jax.numpy as jnp
from jax import lax
from jax.experimental import pallas as pl
from jax.experimental.pallas import tpu as pltpu
from jax.experimental.pallas import tpu_sc as plsc

N = 10000
E = 320000
D = 128
R = 8
V = 10000
T = 16
L = 16
NC = 2
NS = 16
NW = NC * NS
C = D // L
CPS = C // NC
NT = N * T
RN = R * N
CSH = RN // NS
AROWS = CPS * N
GROWS = R * N
TROWS = 1280
NTP = TROWS * 128
EROWS = 2560
EP = EROWS * 128
SROWS_E = EROWS // NS
SROWS_T = TROWS // NS
BLK = 80
NBUF = 8
PD = NBUF - 2

_SC_PARAMS = pltpu.CompilerParams(
    use_tc_tiling_on_sc=False, needs_layout_passes=False)


def _mesh():
    return plsc.VectorSubcoreMesh(
        core_axis_name="c", subcore_axis_name="s", num_cores=NC, num_subcores=NS
    )


def _row_range(total, sid):
    return (total * sid) // NS, (total * (sid + 1)) // NS


def _ring(table, gblk, sblk, rows_v, acc_sh, gsems, ssems):
    dg = {}
    pend = {}
    for j in range(min(PD, BLK)):
        s = j % NBUF
        dg[s] = pltpu.async_copy(table.at[gblk.at[j]], rows_v.at[s], gsems[s])
    for j in range(BLK):
        s = j % NBUF
        dg.pop(s).wait()
        pend[s] = pltpu.async_copy(rows_v.at[s], acc_sh.at[sblk.at[j]],
                                   ssems[s], add=True)
        nj = j + PD
        if nj < BLK:
            s2 = nj % NBUF
            if s2 in pend:
                pend.pop(s2).wait()
            dg[s2] = pltpu.async_copy(table.at[gblk.at[nj]], rows_v.at[s2],
                                      gsems[s2])
    for s2 in list(pend):
        pend.pop(s2).wait()


TBLK = TROWS // NC // NS


def _embed_body(emb_z, xpad_f, psidx, esidx_f, zeros3, zerosf,
                sp_out, cnt_out,
                gblk, sblk, rows_v, cnt_local, acc_sh, gsem, ssem):
    cid = lax.axis_index("c")
    sid = lax.axis_index("s")
    gsems = [gsem.at[i] for i in range(2)]
    ssems = [ssem.at[i] for i in range(2)]

    pltpu.sync_copy(zeros3, rows_v.at[0, pl.ds(0, 125)])
    for i in range(5):
        pltpu.sync_copy(rows_v.at[0, pl.ds(0, 125)],
                        acc_sh.at[pl.ds(625 * sid + 125 * i, 125)])

    @pl.when(sid == 0)
    def _():
        pltpu.sync_copy(rows_v.at[0, pl.ds(0, 16)], acc_sh.at[pl.ds(N, 16)])

    pltpu.sync_copy(zerosf, cnt_local)
    plsc.subcore_barrier()

    ones = jnp.full((L,), 1.0, jnp.float32)
    clo = sid * CSH
    half = EP // NC

    def _cnt_blk(b, carry):
        pltpu.sync_copy(esidx_f.at[pl.ds(cid * half + b * 5120, 5120)], gblk)

        def _cnt(k, c2):
            f = gblk[pl.ds(16 * k, 16)]
            fl = f - clo
            m = (fl >= 0) & (fl < CSH)
            fl = jnp.where(m, fl, 0)
            plsc.addupdate_scatter(cnt_local, [fl], ones, mask=m)
            return c2

        lax.fori_loop(0, 320, _cnt, 0)
        return carry

    lax.fori_loop(0, half // 5120, _cnt_blk, 0)

    row0 = cid * (TROWS // NC) + sid * TBLK
    pltpu.sync_copy(xpad_f.at[pl.ds(row0 * 128, TBLK * 128)], gblk)
    pltpu.sync_copy(psidx.at[pl.ds(row0, TBLK)], sblk)

    dg = {}
    pend = {}
    for j in range(2):
        dg[j] = pltpu.async_copy(
            emb_z.at[gblk.at[pl.ds(128 * j, 128)]], rows_v.at[j], gsems[j])
    for j in range(TBLK):
        s = j % 2
        dg.pop(s).wait()
        pend[s] = pltpu.async_copy(rows_v.at[s], acc_sh.at[sblk.at[j]],
                                   ssems[s], add=True)
        if j + 2 < TBLK:
            pend.pop(s).wait()
            dg[s] = pltpu.async_copy(
                emb_z.at[gblk.at[pl.ds(128 * (j + 2), 128)]], rows_v.at[s],
                gsems[s])
    for s in list(pend):
        pend.pop(s).wait()

    plsc.subcore_barrier()

    for i in range(5):
        base = 625 * sid + 125 * i
        pltpu.sync_copy(acc_sh.at[pl.ds(base, 125)],
                        rows_v.at[0, pl.ds(0, 125)])
        pltpu.sync_copy(rows_v.at[0, pl.ds(0, 125)],
                        sp_out.at[cid, pl.ds(base, 125)])
    pltpu.sync_copy(cnt_local, cnt_out.at[cid, sid])


def _sc_embed():
    return pl.kernel(
        _embed_body,
        out_type=(
            jax.ShapeDtypeStruct((NC, N, D), jnp.float32),
            jax.ShapeDtypeStruct((NC, NS, CSH), jnp.float32),
        ),
        mesh=_mesh(),
        scratch_types=[
            pltpu.VMEM((TBLK * 128,), jnp.int32),
            pltpu.VMEM((TBLK, 128), jnp.int32),
            pltpu.VMEM((2, 128, D), jnp.float32),
            pltpu.VMEM((CSH,), jnp.float32),
            pltpu.MemorySpace.VMEM_SHARED((N + 16, D), jnp.float32),
            pltpu.SemaphoreType.DMA((2,)),
            pltpu.SemaphoreType.DMA((2,)),
        ],
        compiler_params=_SC_PARAMS,
    )


def _agg_body(h_flat, gsrc, esidx, zeros2, agg_out,
              buf, gblk, sblk, rows_v, acc_sh, gsem, ssem):
    cid = lax.axis_index("c")
    sid = lax.axis_index("s")
    gsems = [gsem.at[i] for i in range(NBUF)]
    ssems = [ssem.at[i] for i in range(NBUF)]

    for lc in range(CPS):
        c = cid * CPS + lc
        pltpu.sync_copy(zeros2, buf)
        for i in range(8):
            pltpu.sync_copy(buf, acc_sh.at[pl.ds(5000 * sid + 625 * i, 625)])
        plsc.subcore_barrier()

        for blk in range(SROWS_E // BLK):
            row0 = sid * SROWS_E + blk * BLK
            pltpu.sync_copy(gsrc.at[c, pl.ds(row0, BLK)], gblk)
            pltpu.sync_copy(esidx.at[pl.ds(row0, BLK)], sblk)
            _ring(h_flat, gblk, sblk, rows_v, acc_sh, gsems, ssems)
        plsc.subcore_barrier()

        def _wb(i, carry):
            base = 5000 * sid + 625 * i
            pltpu.sync_copy(acc_sh.at[pl.ds(base, 625)], buf)
            pltpu.sync_copy(buf, agg_out.at[pl.ds(base, 625), c, :])
            return carry

        lax.fori_loop(0, 8, _wb, 0)
        plsc.subcore_barrier()


def _sc_agg():
    return pl.kernel(
        _agg_body,
        out_type=jax.ShapeDtypeStruct((GROWS, C, L), jnp.float32),
        mesh=_mesh(),
        scratch_types=[
            pltpu.VMEM((625, L), jnp.float32),
            pltpu.VMEM((BLK, 128), jnp.int32),
            pltpu.VMEM((BLK, 128), jnp.int32),
            pltpu.VMEM((NBUF, 128, L), jnp.float32),
            pltpu.MemorySpace.VMEM_SHARED((GROWS + 128, L), jnp.float32),
            pltpu.SemaphoreType.DMA((NBUF,)),
            pltpu.SemaphoreType.DMA((NBUF,)),
        ],
        compiler_params=_SC_PARAMS,
    )


BN2 = 2000


def _scale_body(x_ref, s_ref, out_ref):
    mask = (x_ref[...] != 0).astype(jnp.float32)
    den = jnp.sum(mask, axis=1, keepdims=True)
    rec = 1.0 / jnp.maximum(den, 1.0)
    out_ref[...] = (s_ref[0] + s_ref[1]) * rec


def _tc_scale():
    return pl.pallas_call(
        _scale_body,
        grid=(N // BN2,),
        in_specs=[
            pl.BlockSpec((BN2, T), lambda i: (i, 0)),
            pl.BlockSpec((NC, BN2, D), lambda i: (0, i, 0)),
        ],
        out_specs=pl.BlockSpec((BN2, D), lambda i: (i, 0)),
        out_shape=jax.ShapeDtypeStruct((N, D), jnp.float32),
    )


BN = 400


def _combine_body(h_ref, agg_ref, cnt_ref, wrel_ref, wroot_ref, b_ref,
                  res_ref, out_ref):
    h = h_ref[...]
    acc = jnp.dot(h, wroot_ref[...], preferred_element_type=jnp.float32)
    acc = acc + b_ref[...]
    cnt = jnp.sum(cnt_ref[...].reshape(BN, NC, R), axis=1)
    recip = 1.0 / jnp.maximum(cnt, 1.0)
    for r in range(R):
        ar = agg_ref[r] * recip[:, r][:, None]
        acc = acc + jnp.dot(ar, wrel_ref[r], preferred_element_type=jnp.float32)
    acc = jnp.maximum(acc, 0.0)
    if res_ref is not None:
        acc = acc + res_ref[...]
    out_ref[...] = acc


def _tc_combine(with_res):
    body = _combine_body if with_res else (
        lambda h, a, c, wr, wo, b, o: _combine_body(h, a, c, wr, wo, b, None, o)
    )
    in_specs = [
        pl.BlockSpec((BN, D), lambda i: (i, 0)),
        pl.BlockSpec((R, BN, D), lambda i: (0, i, 0)),
        pl.BlockSpec((BN, NC * R), lambda i: (i, 0)),
        pl.BlockSpec((R, D, D), lambda i: (0, 0, 0)),
        pl.BlockSpec((D, D), lambda i: (0, 0)),
        pl.BlockSpec((1, D), lambda i: (0, 0)),
    ]
    if with_res:
        in_specs.append(pl.BlockSpec((BN, D), lambda i: (i, 0)))
    return pl.pallas_call(
        body,
        grid=(N // BN,),
        in_specs=in_specs,
        out_specs=pl.BlockSpec((BN, D), lambda i: (i, 0)),
        out_shape=jax.ShapeDtypeStruct((N, D), jnp.float32),
    )


def _perm(h):
    return h.reshape(N, C, L).transpose(1, 0, 2).reshape(C * N, L)


def _unperm(hp):
    return hp.reshape(C, N, L).transpose(1, 0, 2).reshape(N, D)


def _unperm_agg(agg_out):
    return agg_out.reshape(C, R, N, L).transpose(1, 2, 0, 3).reshape(R, N, D)


def kernel(x, edge_index, edge_type, emb, W_rel0, W_root0, b0,
           W_rel1, W_root1, b1):
    x = x.astype(jnp.int32)
    src = edge_index[0].astype(jnp.int32)
    dst = edge_index[1].astype(jnp.int32)
    et = edge_type.astype(jnp.int32)

    emb_z = emb.at[0].set(0.0)
    xpad = jnp.concatenate(
        [x.reshape(NT), jnp.zeros((NTP - NT,), jnp.int32)]
    ).reshape(TROWS, 128)
    psidx = jnp.concatenate(
        [jnp.arange(NT, dtype=jnp.int32) // T,
         jnp.full((NTP - NT,), N, jnp.int32)]).reshape(TROWS, 128)
    esidx_f = jnp.concatenate(
        [et * N + dst, jnp.full((EP - E,), GROWS, jnp.int32)])
    esidx = esidx_f.reshape(EROWS, 128)
    src_p = jnp.concatenate([src, jnp.zeros((EP - E,), jnp.int32)])
    gsrc = (src_p[None, :] * C + jnp.arange(C, dtype=jnp.int32)[:, None]
            ).reshape(C, EROWS, 128)
    zeros2 = jnp.zeros((625, L), jnp.float32)
    zeros3 = jnp.zeros((125, D), jnp.float32)
    zerosf = jnp.zeros((CSH,), jnp.float32)
    b0r = b0.reshape(1, D)
    b1r = b1.reshape(1, D)

    sp_p, cnt_raw = _sc_embed()(emb_z, xpad.reshape(NTP), psidx,
                                esidx_f, zeros3, zerosf)
    cnt_t = cnt_raw.reshape(NC, R, N).transpose(2, 0, 1).reshape(N, NC * R)

    h0 = _tc_scale()(x, sp_p)

    agg0 = _sc_agg()(h0.reshape(N * C, L), gsrc, esidx, zeros2
                     ).reshape(R, N, D)
    out0 = _tc_combine(False)(h0, agg0, cnt_t, W_rel0, W_root0, b0r)

    agg1 = _sc_agg()(out0.reshape(N * C, L), gsrc, esidx, zeros2
                     ).reshape(R, N, D)
    out = _tc_combine(True)(out0, agg1, cnt_t, W_rel1, W_root1, b1r, out0)
    return out


# speedup vs baseline: 1.0304x; 1.0166x over previous
import functools
import numpy as np
import jax
import jax.numpy as jnp
from jax import lax
from jax.experimental import pallas as pl
from jax.experimental.pallas import tpu as pltpu
from jax.experimental.pallas import tpu_sc as plsc

N = 10000
E = 320000
D = 128
R = 8
V = 10000
T = 16
L = 16
NC = 2
NS = 16
NW = NC * NS
C = D // L
CPS = C // NC
NT = N * T
RN = R * N
CSH = RN // NS
AROWS = CPS * N
GROWS = R * N
TROWS = 1280
NTP = TROWS * 128
EROWS = 2560
EP = EROWS * 128
SROWS_E = EROWS // NS
SROWS_T = TROWS // NS
BLK = 80
NBUF = 9
PD = NBUF - 2

_SC_PARAMS = pltpu.CompilerParams(
    use_tc_tiling_on_sc=False, needs_layout_passes=False)


def _mesh():
    return plsc.VectorSubcoreMesh(
        core_axis_name="c", subcore_axis_name="s", num_cores=NC, num_subcores=NS
    )


def _row_range(total, sid):
    return (total * sid) // NS, (total * (sid + 1)) // NS


def _ring(table, gblk, sblk, rows_v, acc_sh, gsems, ssems):
    dg = {}
    pend = {}
    for j in range(min(PD, BLK)):
        s = j % NBUF
        dg[s] = pltpu.async_copy(table.at[gblk.at[j]], rows_v.at[s], gsems[s])
    for j in range(BLK):
        s = j % NBUF
        dg.pop(s).wait()
        pend[s] = pltpu.async_copy(rows_v.at[s], acc_sh.at[sblk.at[j]],
                                   ssems[s], add=True)
        nj = j + PD
        if nj < BLK:
            s2 = nj % NBUF
            if s2 in pend:
                pend.pop(s2).wait()
            dg[s2] = pltpu.async_copy(table.at[gblk.at[nj]], rows_v.at[s2],
                                      gsems[s2])
    for s2 in list(pend):
        pend.pop(s2).wait()


TBLK = TROWS // NC // NS


def _embed_body(emb_z, xpad_f, psidx, esidx_f, zeros3, zerosf,
                sp_out, cnt_out,
                gblk, sblk, rows_v, cnt_local, acc_sh, gsem, ssem):
    cid = lax.axis_index("c")
    sid = lax.axis_index("s")
    gsems = [gsem.at[i] for i in range(2)]
    ssems = [ssem.at[i] for i in range(2)]

    pltpu.sync_copy(zeros3, rows_v.at[0, pl.ds(0, 125)])
    for i in range(5):
        pltpu.sync_copy(rows_v.at[0, pl.ds(0, 125)],
                        acc_sh.at[pl.ds(625 * sid + 125 * i, 125)])

    @pl.when(sid == 0)
    def _():
        pltpu.sync_copy(rows_v.at[0, pl.ds(0, 16)], acc_sh.at[pl.ds(N, 16)])

    pltpu.sync_copy(zerosf, cnt_local)
    plsc.subcore_barrier()

    ones = jnp.full((L,), 1.0, jnp.float32)
    clo = sid * CSH
    half = EP // NC

    def _cnt_blk(b, carry):
        pltpu.sync_copy(esidx_f.at[pl.ds(cid * half + b * 5120, 5120)], gblk)

        def _cnt(k, c2):
            for u in range(8):
                f = gblk[pl.ds(128 * k + 16 * u, 16)]
                fl = f - clo
                m = (fl >= 0) & (fl < CSH)
                fl = jnp.where(m, fl, 0)
                plsc.addupdate_scatter(cnt_local, [fl], ones, mask=m)
            return c2

        lax.fori_loop(0, 40, _cnt, 0)
        return carry

    lax.fori_loop(0, half // 5120, _cnt_blk, 0)

    row0 = cid * (TROWS // NC) + sid * TBLK
    pltpu.sync_copy(xpad_f.at[pl.ds(row0 * 128, TBLK * 128)], gblk)
    pltpu.sync_copy(psidx.at[pl.ds(row0, TBLK)], sblk)

    dg = {}
    pend = {}
    for j in range(2):
        dg[j] = pltpu.async_copy(
            emb_z.at[gblk.at[pl.ds(128 * j, 128)]], rows_v.at[j], gsems[j])
    for j in range(TBLK):
        s = j % 2
        dg.pop(s).wait()
        pend[s] = pltpu.async_copy(rows_v.at[s], acc_sh.at[sblk.at[j]],
                                   ssems[s], add=True)
        if j + 2 < TBLK:
            pend.pop(s).wait()
            dg[s] = pltpu.async_copy(
                emb_z.at[gblk.at[pl.ds(128 * (j + 2), 128)]], rows_v.at[s],
                gsems[s])
    for s in list(pend):
        pend.pop(s).wait()

    plsc.subcore_barrier()

    for i in range(5):
        base = 625 * sid + 125 * i
        pltpu.sync_copy(acc_sh.at[pl.ds(base, 125)],
                        rows_v.at[0, pl.ds(0, 125)])
        pltpu.sync_copy(rows_v.at[0, pl.ds(0, 125)],
                        sp_out.at[cid, pl.ds(base, 125)])
    pltpu.sync_copy(cnt_local, cnt_out.at[cid, sid])


def _sc_embed():
    return pl.kernel(
        _embed_body,
        out_type=(
            jax.ShapeDtypeStruct((NC, N, D), jnp.float32),
            jax.ShapeDtypeStruct((NC, NS, CSH), jnp.float32),
        ),
        mesh=_mesh(),
        scratch_types=[
            pltpu.VMEM((TBLK * 128,), jnp.int32),
            pltpu.VMEM((TBLK, 128), jnp.int32),
            pltpu.VMEM((2, 128, D), jnp.float32),
            pltpu.VMEM((CSH,), jnp.float32),
            pltpu.MemorySpace.VMEM_SHARED((N + 16, D), jnp.float32),
            pltpu.SemaphoreType.DMA((2,)),
            pltpu.SemaphoreType.DMA((2,)),
        ],
        compiler_params=_SC_PARAMS,
    )


def _agg_body(h_flat, gsrc, esidx, zeros2, agg_out,
              buf, gblk, sblk, rows_v, acc_sh, gsem, ssem):
    cid = lax.axis_index("c")
    sid = lax.axis_index("s")
    gsems = [gsem.at[i] for i in range(NBUF)]
    ssems = [ssem.at[i] for i in range(NBUF)]

    for lc in range(CPS):
        c = cid * CPS + lc
        pltpu.sync_copy(zeros2, buf)
        for i in range(8):
            pltpu.sync_copy(buf, acc_sh.at[pl.ds(5000 * sid + 625 * i, 625)])
        plsc.subcore_barrier()

        for blk in range(SROWS_E // BLK):
            row0 = sid * SROWS_E + blk * BLK
            pltpu.sync_copy(gsrc.at[c, pl.ds(row0, BLK)], gblk)
            pltpu.sync_copy(esidx.at[pl.ds(row0, BLK)], sblk)
            _ring(h_flat, gblk, sblk, rows_v, acc_sh, gsems, ssems)
        plsc.subcore_barrier()

        def _wb(i, carry):
            base = 5000 * sid + 625 * i
            pltpu.sync_copy(acc_sh.at[pl.ds(base, 625)], buf)
            pltpu.sync_copy(buf, agg_out.at[pl.ds(base, 625), c, :])
            return carry

        lax.fori_loop(0, 8, _wb, 0)
        plsc.subcore_barrier()


def _sc_agg():
    return pl.kernel(
        _agg_body,
        out_type=jax.ShapeDtypeStruct((GROWS, C, L), jnp.float32),
        mesh=_mesh(),
        scratch_types=[
            pltpu.VMEM((625, L), jnp.float32),
            pltpu.VMEM((BLK, 128), jnp.int32),
            pltpu.VMEM((BLK, 128), jnp.int32),
            pltpu.VMEM((NBUF, 128, L), jnp.float32),
            pltpu.MemorySpace.VMEM_SHARED((GROWS + 128, L), jnp.float32),
            pltpu.SemaphoreType.DMA((NBUF,)),
            pltpu.SemaphoreType.DMA((NBUF,)),
        ],
        compiler_params=_SC_PARAMS,
    )


BN2 = 2000


def _scale_body(x_ref, s_ref, out_ref):
    mask = (x_ref[...] != 0).astype(jnp.float32)
    den = jnp.sum(mask, axis=1, keepdims=True)
    rec = 1.0 / jnp.maximum(den, 1.0)
    out_ref[...] = (s_ref[0] + s_ref[1]) * rec


def _tc_scale():
    return pl.pallas_call(
        _scale_body,
        grid=(N // BN2,),
        in_specs=[
            pl.BlockSpec((BN2, T), lambda i: (i, 0)),
            pl.BlockSpec((NC, BN2, D), lambda i: (0, i, 0)),
        ],
        out_specs=pl.BlockSpec((BN2, D), lambda i: (i, 0)),
        out_shape=jax.ShapeDtypeStruct((N, D), jnp.float32),
    )


BN = 400


def _combine_body(h_ref, agg_ref, cnt_ref, wrel_ref, wroot_ref, b_ref,
                  res_ref, out_ref):
    h = h_ref[...]
    acc = jnp.dot(h, wroot_ref[...], preferred_element_type=jnp.float32)
    acc = acc + b_ref[...]
    cnt = jnp.sum(cnt_ref[...].reshape(BN, NC, R), axis=1)
    recip = 1.0 / jnp.maximum(cnt, 1.0)
    for r in range(R):
        ar = agg_ref[r] * recip[:, r][:, None]
        acc = acc + jnp.dot(ar, wrel_ref[r], preferred_element_type=jnp.float32)
    acc = jnp.maximum(acc, 0.0)
    if res_ref is not None:
        acc = acc + res_ref[...]
    out_ref[...] = acc


def _tc_combine(with_res):
    body = _combine_body if with_res else (
        lambda h, a, c, wr, wo, b, o: _combine_body(h, a, c, wr, wo, b, None, o)
    )
    in_specs = [
        pl.BlockSpec((BN, D), lambda i: (i, 0)),
        pl.BlockSpec((R, BN, D), lambda i: (0, i, 0)),
        pl.BlockSpec((BN, NC * R), lambda i: (i, 0)),
        pl.BlockSpec((R, D, D), lambda i: (0, 0, 0)),
        pl.BlockSpec((D, D), lambda i: (0, 0)),
        pl.BlockSpec((1, D), lambda i: (0, 0)),
    ]
    if with_res:
        in_specs.append(pl.BlockSpec((BN, D), lambda i: (i, 0)))
    return pl.pallas_call(
        body,
        grid=(N // BN,),
        in_specs=in_specs,
        out_specs=pl.BlockSpec((BN, D), lambda i: (i, 0)),
        out_shape=jax.ShapeDtypeStruct((N, D), jnp.float32),
    )


def _perm(h):
    return h.reshape(N, C, L).transpose(1, 0, 2).reshape(C * N, L)


def _unperm(hp):
    return hp.reshape(C, N, L).transpose(1, 0, 2).reshape(N, D)


def _unperm_agg(agg_out):
    return agg_out.reshape(C, R, N, L).transpose(1, 2, 0, 3).reshape(R, N, D)


def kernel(x, edge_index, edge_type, emb, W_rel0, W_root0, b0,
           W_rel1, W_root1, b1):
    x = x.astype(jnp.int32)
    src = edge_index[0].astype(jnp.int32)
    dst = edge_index[1].astype(jnp.int32)
    et = edge_type.astype(jnp.int32)

    emb_z = emb.at[0].set(0.0)
    xpad = jnp.concatenate(
        [x.reshape(NT), jnp.zeros((NTP - NT,), jnp.int32)]
    ).reshape(TROWS, 128)
    psidx = jnp.concatenate(
        [jnp.arange(NT, dtype=jnp.int32) // T,
         jnp.full((NTP - NT,), N, jnp.int32)]).reshape(TROWS, 128)
    esidx_f = jnp.concatenate(
        [et * N + dst, jnp.full((EP - E,), GROWS, jnp.int32)])
    esidx = esidx_f.reshape(EROWS, 128)
    src_p = jnp.concatenate([src, jnp.zeros((EP - E,), jnp.int32)])
    gsrc = (src_p[None, :] * C + jnp.arange(C, dtype=jnp.int32)[:, None]
            ).reshape(C, EROWS, 128)
    zeros2 = jnp.zeros((625, L), jnp.float32)
    zeros3 = jnp.zeros((125, D), jnp.float32)
    zerosf = jnp.zeros((CSH,), jnp.float32)
    b0r = b0.reshape(1, D)
    b1r = b1.reshape(1, D)

    sp_p, cnt_raw = _sc_embed()(emb_z, xpad.reshape(NTP), psidx,
                                esidx_f, zeros3, zerosf)
    cnt_t = cnt_raw.reshape(NC, R, N).transpose(2, 0, 1).reshape(N, NC * R)

    h0 = _tc_scale()(x, sp_p)

    agg0 = _sc_agg()(h0.reshape(N * C, L), gsrc, esidx, zeros2
                     ).reshape(R, N, D)
    out0 = _tc_combine(False)(h0, agg0, cnt_t, W_rel0, W_root0, b0r)

    agg1 = _sc_agg()(out0.reshape(N * C, L), gsrc, esidx, zeros2
                     ).reshape(R, N, D)
    out = _tc_combine(True)(out0, agg1, cnt_t, W_rel1, W_root1, b1r, out0)
    return out
